# baseline (speedup 1.0000x reference)
import functools

import jax
import jax.numpy as jnp
from jax import lax
from jax.experimental import pallas as pl
from jax.experimental.pallas import tpu as pltpu

B, T, D, H = 1, 2048, 768, 12
E, K, F = 8, 2, 3072
DH = D // H
NT = B * T
BM = 128
P = NT * K + E * BM
BQ = 1024
BR = 256
NEG = -1e30


def _ln_rows(x, g, b):
    m = jnp.mean(x, axis=-1, keepdims=True)
    v = jnp.mean((x - m) * (x - m), axis=-1, keepdims=True)
    return (x - m) / jnp.sqrt(v + 1e-5) * g + b


def _ln_qkv_body(x_ref, g_ref, b_ref, wq_ref, wk_ref, wv_ref,
                 q_ref, k_ref, v_ref):
    h = _ln_rows(x_ref[...], g_ref[...], b_ref[...])
    q_ref[...] = jnp.dot(h, wq_ref[...], preferred_element_type=jnp.float32)
    k_ref[...] = jnp.dot(h, wk_ref[...], preferred_element_type=jnp.float32)
    v_ref[...] = jnp.dot(h, wv_ref[...], preferred_element_type=jnp.float32)


def _ln_qkv(x, g, b, wq, wk, wv):
    return pl.pallas_call(
        _ln_qkv_body,
        grid=(NT // BR,),
        in_specs=[
            pl.BlockSpec((BR, D), lambda i: (i, 0)),
            pl.BlockSpec((1, D), lambda i: (0, 0)),
            pl.BlockSpec((1, D), lambda i: (0, 0)),
            pl.BlockSpec((D, D), lambda i: (0, 0)),
            pl.BlockSpec((D, D), lambda i: (0, 0)),
            pl.BlockSpec((D, D), lambda i: (0, 0)),
        ],
        out_specs=[pl.BlockSpec((BR, D), lambda i: (i, 0))] * 3,
        out_shape=[jax.ShapeDtypeStruct((NT, D), jnp.float32)] * 3,
    )(x, g, b, wq, wk, wv)


def _attn_body(q_ref, k_ref, v_ref, o_ref, s_ref):
    i = pl.program_id(0)
    nc = i + 1
    row = i * BQ + lax.broadcasted_iota(jnp.int32, (BQ, BQ), 0)

    def one_head(hh):
        q = q_ref[:, hh * DH:(hh + 1) * DH]

        def score_chunk(c, _):
            k = k_ref[pl.ds(c * BQ, BQ), hh * DH:(hh + 1) * DH]
            s = lax.dot_general(q, k, (((1,), (1,)), ((), ())),
                                preferred_element_type=jnp.float32)
            s = s * (1.0 / (DH ** 0.5))
            col = c * BQ + lax.broadcasted_iota(jnp.int32, (BQ, BQ), 1)
            s_ref[:, pl.ds(c * BQ, BQ)] = jnp.where(col <= row, s, -1e9)
            return 0

        lax.fori_loop(0, nc, score_chunk, 0)
        s = s_ref[...]
        col = lax.broadcasted_iota(jnp.int32, s.shape, 1)
        s = jnp.where(col < nc * BQ, s, -1e9)
        m = jnp.max(s, axis=-1, keepdims=True)
        p = jnp.exp(s - m)
        rcp = 1.0 / jnp.sum(p, axis=-1, keepdims=True)
        s_ref[...] = p

        def av_chunk(c, acc):
            pc = s_ref[:, pl.ds(c * BQ, BQ)]
            vc = v_ref[pl.ds(c * BQ, BQ), hh * DH:(hh + 1) * DH]
            return acc + jnp.dot(pc, vc, preferred_element_type=jnp.float32)

        acc = lax.fori_loop(0, nc, av_chunk, jnp.zeros((BQ, DH), jnp.float32))
        return acc * rcp

    for hh in range(H):
        o_ref[:, hh * DH:(hh + 1) * DH] = one_head(hh)


def _attention(q, k, v):
    return pl.pallas_call(
        _attn_body,
        grid=(T // BQ,),
        in_specs=[
            pl.BlockSpec((BQ, D), lambda i: (i, 0)),
            pl.BlockSpec((T, D), lambda i: (0, 0)),
            pl.BlockSpec((T, D), lambda i: (0, 0)),
        ],
        out_specs=pl.BlockSpec((BQ, D), lambda i: (i, 0)),
        out_shape=jax.ShapeDtypeStruct((NT, D), jnp.float32),
        scratch_shapes=[pltpu.VMEM((BQ, T), jnp.float32)],
    )(q, k, v)


def _ln2_router_body(a_ref, wo_ref, x_ref, g_ref, b_ref, wr_ref, xr_ref,
                     h2_ref, i1_ref, i2_ref, w1_ref, w2_ref):
    xr = x_ref[...] + jnp.dot(a_ref[...], wo_ref[...],
                              preferred_element_type=jnp.float32)
    xr_ref[...] = xr
    h2 = _ln_rows(xr, g_ref[...], b_ref[...])
    h2_ref[...] = h2
    lg = jnp.dot(h2, wr_ref[...], preferred_element_type=jnp.float32)
    lane = lax.broadcasted_iota(jnp.int32, lg.shape, 1)
    lg = jnp.where(lane < E, lg, NEG)
    m1 = jnp.max(lg, axis=-1, keepdims=True)
    i1 = jnp.min(jnp.where(lg >= m1, lane, E), axis=-1, keepdims=True)
    lg2 = jnp.where(lane == i1, NEG, lg)
    m2 = jnp.max(lg2, axis=-1, keepdims=True)
    i2 = jnp.min(jnp.where(lg2 >= m2, lane, E), axis=-1, keepdims=True)
    w2 = 1.0 / (1.0 + jnp.exp(m1 - m2))
    i1_ref[...] = i1
    i2_ref[...] = i2
    w1_ref[...] = 1.0 - w2
    w2_ref[...] = w2


def _ln2_router(attn, wo, x, g, b, wr_pad):
    return pl.pallas_call(
        _ln2_router_body,
        grid=(NT // BR,),
        in_specs=[
            pl.BlockSpec((BR, D), lambda i: (i, 0)),
            pl.BlockSpec((D, D), lambda i: (0, 0)),
            pl.BlockSpec((BR, D), lambda i: (i, 0)),
            pl.BlockSpec((1, D), lambda i: (0, 0)),
            pl.BlockSpec((1, D), lambda i: (0, 0)),
            pl.BlockSpec((D, 128), lambda i: (0, 0)),
        ],
        out_specs=[
            pl.BlockSpec((BR, D), lambda i: (i, 0)),
            pl.BlockSpec((BR, D), lambda i: (i, 0)),
            pl.BlockSpec((BR, 1), lambda i: (i, 0)),
            pl.BlockSpec((BR, 1), lambda i: (i, 0)),
            pl.BlockSpec((BR, 1), lambda i: (i, 0)),
            pl.BlockSpec((BR, 1), lambda i: (i, 0)),
        ],
        out_shape=[
            jax.ShapeDtypeStruct((NT, D), jnp.float32),
            jax.ShapeDtypeStruct((NT, D), jnp.float32),
            jax.ShapeDtypeStruct((NT, 1), jnp.int32),
            jax.ShapeDtypeStruct((NT, 1), jnp.int32),
            jax.ShapeDtypeStruct((NT, 1), jnp.float32),
            jax.ShapeDtypeStruct((NT, 1), jnp.float32),
        ],
    )(attn, wo, x, g, b, wr_pad)


def _route_rank_body(a_ref, slot_ref, cnt_ref):
    a = a_ref[...]
    f32 = jnp.float32
    MU = (lax.broadcasted_iota(jnp.int32, (128, 128), 0) <
          lax.broadcasted_iota(jnp.int32, (128, 128), 1)).astype(f32)
    NL = (lax.broadcasted_iota(jnp.int32, (32, 32), 1) <
          lax.broadcasted_iota(jnp.int32, (32, 32), 0)).astype(f32)
    ONES = jnp.ones((128, 128), f32)
    slot = jnp.zeros(a.shape, f32)
    seg_base = jnp.zeros((), f32)
    for e in range(E):
        mask = (a == e).astype(f32)
        pre = jnp.dot(mask, MU, preferred_element_type=f32)
        rsl = jnp.dot(mask, ONES, preferred_element_type=f32)
        exr = jnp.dot(NL, rsl, preferred_element_type=f32)
        rank = pre + exr
        slot = jnp.where(a == e, seg_base + rank, slot)
        cnt = jnp.sum(mask)
        cnt_ref[e:e + 1, :] = jnp.full((1, 128), cnt, f32)
        padded = jnp.ceil(cnt * (1.0 / BM)) * BM
        seg_base = seg_base + padded
    slot_ref[...] = slot.astype(jnp.int32)


def _route_rank(aflat2d):
    return pl.pallas_call(
        _route_rank_body,
        in_specs=[pl.BlockSpec((32, 128), lambda: (0, 0))],
        out_specs=[pl.BlockSpec((32, 128), lambda: (0, 0)),
                   pl.BlockSpec((E, 128), lambda: (0, 0))],
        out_shape=[jax.ShapeDtypeStruct((32, 128), jnp.int32),
                   jax.ShapeDtypeStruct((E, 128), jnp.float32)],
        grid=(),
    )(aflat2d)


def _sc_scatter_rows(table, idx, n_out):
    from jax.experimental.pallas import tpu_sc as plsc
    NB = idx.shape[0]
    NW = 32
    b_per_w = NB // NW
    mesh = plsc.VectorSubcoreMesh(core_axis_name="c", subcore_axis_name="s")

    @functools.partial(
        pl.kernel, mesh=mesh,
        out_type=jax.ShapeDtypeStruct((n_out, D), jnp.float32),
        scratch_types=[
            pltpu.VMEM((b_per_w,), jnp.int32),
            pltpu.VMEM((b_per_w, D), jnp.float32),
            pltpu.SemaphoreType.DMA,
        ],
    )
    def k(table_hbm, idx_hbm, out_hbm, idx_v, rows_v, sem):
        wid = lax.axis_index("s") * 2 + lax.axis_index("c")
        pltpu.sync_copy(idx_hbm.at[pl.ds(wid * b_per_w, b_per_w)], idx_v)
        t0 = (wid % (NT // b_per_w)) * b_per_w
        pltpu.sync_copy(table_hbm.at[pl.ds(t0, b_per_w)], rows_v)
        pltpu.async_copy(rows_v, out_hbm.at[idx_v], sem).wait()

    return k(table, idx)


def _moe_body(te_ref, perm_ref, w1_ref, w2_ref, o_ref):
    a = jnp.dot(perm_ref[...], w1_ref[0], preferred_element_type=jnp.float32)
    a = 0.5 * a * (1.0 + lax.erf(a * (2.0 ** -0.5)))
    o_ref[...] = jnp.dot(a, w2_ref[0], preferred_element_type=jnp.float32)


def _moe_ffn(tile_expert, perm, w1, w2):
    grid_spec = pltpu.PrefetchScalarGridSpec(
        num_scalar_prefetch=1,
        grid=(P // BM,),
        in_specs=[
            pl.BlockSpec((BM, D), lambda i, te: (i, 0)),
            pl.BlockSpec((1, D, F), lambda i, te: (te[i], 0, 0)),
            pl.BlockSpec((1, F, D), lambda i, te: (te[i], 0, 0)),
        ],
        out_specs=pl.BlockSpec((BM, D), lambda i, te: (i, 0)),
    )
    return pl.pallas_call(
        _moe_body,
        grid_spec=grid_spec,
        out_shape=jax.ShapeDtypeStruct((P, D), jnp.float32),
    )(tile_expert, perm, w1, w2)


def _sc_gather(table, idx):
    from jax.experimental.pallas import tpu_sc as plsc
    NB = idx.shape[0]
    NW = 32
    b_per_w = NB // NW
    CH = 128 if b_per_w % 128 == 0 else 80
    nch = b_per_w // CH
    mesh = plsc.VectorSubcoreMesh(core_axis_name="c", subcore_axis_name="s")

    @functools.partial(
        pl.kernel, mesh=mesh,
        out_type=jax.ShapeDtypeStruct((NB, D), jnp.float32),
        scratch_types=[
            pltpu.VMEM((b_per_w,), jnp.int32),
            pltpu.VMEM((b_per_w, D), jnp.float32),
            pltpu.SemaphoreType.DMA,
        ],
    )
    def k(table_hbm, idx_hbm, out_hbm, idx_v, rows_v, sem):
        wid = lax.axis_index("s") * 2 + lax.axis_index("c")
        base = wid * b_per_w
        pltpu.sync_copy(idx_hbm.at[pl.ds(base, b_per_w)], idx_v)
        descs = [
            pltpu.async_copy(
                table_hbm.at[idx_v.at[pl.ds(j * CH, CH)]],
                rows_v.at[pl.ds(j * CH, CH)], sem)
            for j in range(nch)
        ]
        for d in descs:
            d.wait()
        pltpu.sync_copy(rows_v, out_hbm.at[pl.ds(base, b_per_w)])

    return k(table, idx)


def _combine_body(x_ref, g1_ref, g2_ref, w1_ref, w2_ref, o_ref):
    o_ref[...] = (x_ref[...] + g1_ref[...] * w1_ref[...]
                  + g2_ref[...] * w2_ref[...])


def _combine(x, g1, g2, wt1, wt2):
    return pl.pallas_call(
        _combine_body,
        grid=(NT // BR,),
        in_specs=[pl.BlockSpec((BR, D), lambda i: (i, 0))] * 3
        + [pl.BlockSpec((BR, 1), lambda i: (i, 0))] * 2,
        out_specs=pl.BlockSpec((BR, D), lambda i: (i, 0)),
        out_shape=jax.ShapeDtypeStruct((NT, D), jnp.float32),
    )(x, g1, g2, wt1, wt2)


def kernel(x, Wq, Wk, Wv, Wo, ln1_g, ln1_b, ln2_g, ln2_b, router_w, w1, w2):
    xf = x.reshape(NT, D)
    q, k, v = _ln_qkv(xf, ln1_g.reshape(1, D), ln1_b.reshape(1, D),
                      Wq, Wk, Wv)
    attn = _attention(q, k, v)

    wr_pad = jnp.zeros((D, 128), jnp.float32).at[:, :E].set(router_w.T)
    xr, h2, i1, i2, wt1, wt2 = _ln2_router(attn, Wo, xf, ln2_g.reshape(1, D),
                                           ln2_b.reshape(1, D), wr_pad)

    i32 = jnp.int32
    aflat2d = jnp.concatenate([i1, i2], axis=0).reshape(32, 128)
    slot2d, cnt2d = _route_rank(aflat2d)
    slot = slot2d.reshape(K * NT)
    counts = cnt2d[:, 0].astype(i32)
    seg_end = jnp.cumsum(((counts + BM - 1) // BM) * BM)
    ntiles = P // BM
    tile_expert = jnp.minimum(
        E - 1,
        jnp.searchsorted(seg_end, jnp.arange(ntiles, dtype=i32) * BM,
                         side='right')).astype(i32)

    perm = _sc_scatter_rows(h2, slot, P)
    eo = _moe_ffn(tile_expert, perm, w1, w2)
    g = _sc_gather(eo, slot)
    return _combine(xr, g[:NT], g[NT:], wt1, wt2).reshape(B, T, D)


# speedup vs baseline: 1.1175x; 1.1175x over previous
import functools

import jax
import jax.numpy as jnp
from jax import lax
from jax.experimental import pallas as pl
from jax.experimental.pallas import tpu as pltpu

B, T, D, H = 1, 2048, 768, 12
E, K, F = 8, 2, 3072
DH = D // H
NT = B * T
BM = 128
P = NT * K + E * BM
BQ = 1024
BR = 256
NEG = -1e30


def _ln_rows(x, g, b):
    m = jnp.mean(x, axis=-1, keepdims=True)
    v = jnp.mean((x - m) * (x - m), axis=-1, keepdims=True)
    return (x - m) / jnp.sqrt(v + 1e-5) * g + b


def _ln_qkv_body(x_ref, g_ref, b_ref, wq_ref, wk_ref, wv_ref,
                 q_ref, k_ref, v_ref):
    h = _ln_rows(x_ref[...], g_ref[...], b_ref[...])
    q_ref[...] = jnp.dot(h, wq_ref[...], preferred_element_type=jnp.float32)
    k_ref[...] = jnp.dot(h, wk_ref[...], preferred_element_type=jnp.float32)
    v_ref[...] = jnp.dot(h, wv_ref[...], preferred_element_type=jnp.float32)


def _ln_qkv(x, g, b, wq, wk, wv):
    return pl.pallas_call(
        _ln_qkv_body,
        grid=(NT // BR,),
        in_specs=[
            pl.BlockSpec((BR, D), lambda i: (i, 0)),
            pl.BlockSpec((1, D), lambda i: (0, 0)),
            pl.BlockSpec((1, D), lambda i: (0, 0)),
            pl.BlockSpec((D, D), lambda i: (0, 0)),
            pl.BlockSpec((D, D), lambda i: (0, 0)),
            pl.BlockSpec((D, D), lambda i: (0, 0)),
        ],
        out_specs=[pl.BlockSpec((BR, D), lambda i: (i, 0))] * 3,
        out_shape=[jax.ShapeDtypeStruct((NT, D), jnp.float32)] * 3,
    )(x, g, b, wq, wk, wv)


def _attn_body(q_ref, k_ref, v_ref, o_ref, s_ref):
    i = pl.program_id(1)
    nc = i + 1
    row = i * BQ + lax.broadcasted_iota(jnp.int32, (BQ, BQ), 0)

    def one_head(hh):
        q = q_ref[:, hh * DH:(hh + 1) * DH]

        def score_chunk(c, _):
            k = k_ref[pl.ds(c * BQ, BQ), hh * DH:(hh + 1) * DH]
            s = lax.dot_general(q, k, (((1,), (1,)), ((), ())),
                                preferred_element_type=jnp.float32)
            s = s * (1.0 / (DH ** 0.5))
            col = c * BQ + lax.broadcasted_iota(jnp.int32, (BQ, BQ), 1)
            s_ref[:, pl.ds(c * BQ, BQ)] = jnp.where(col <= row, s, -1e9)
            return 0

        lax.fori_loop(0, nc, score_chunk, 0)
        s = s_ref[...]
        col = lax.broadcasted_iota(jnp.int32, s.shape, 1)
        s = jnp.where(col < nc * BQ, s, -1e9)
        m = jnp.max(s, axis=-1, keepdims=True)
        p = jnp.exp(s - m)
        rcp = 1.0 / jnp.sum(p, axis=-1, keepdims=True)
        s_ref[...] = p

        def av_chunk(c, acc):
            pc = s_ref[:, pl.ds(c * BQ, BQ)]
            vc = v_ref[pl.ds(c * BQ, BQ), hh * DH:(hh + 1) * DH]
            return acc + jnp.dot(pc, vc, preferred_element_type=jnp.float32)

        acc = lax.fori_loop(0, nc, av_chunk, jnp.zeros((BQ, DH), jnp.float32))
        return acc * rcp

    o_ref[:, :DH] = one_head(0)
    o_ref[:, DH:] = one_head(1)


def _attention(q, k, v):
    return pl.pallas_call(
        _attn_body,
        grid=(H // 2, T // BQ),
        in_specs=[
            pl.BlockSpec((BQ, 128), lambda j, i: (i, j)),
            pl.BlockSpec((T, 128), lambda j, i: (0, j)),
            pl.BlockSpec((T, 128), lambda j, i: (0, j)),
        ],
        out_specs=pl.BlockSpec((BQ, 128), lambda j, i: (i, j)),
        out_shape=jax.ShapeDtypeStruct((NT, D), jnp.float32),
        scratch_shapes=[pltpu.VMEM((BQ, T), jnp.float32)],
    )(q, k, v)


def _ln2_router_body(a_ref, wo_ref, x_ref, g_ref, b_ref, wr_ref, xr_ref,
                     h2_ref, i1_ref, i2_ref, w1_ref, w2_ref):
    xr = x_ref[...] + jnp.dot(a_ref[...], wo_ref[...],
                              preferred_element_type=jnp.float32)
    xr_ref[...] = xr
    h2 = _ln_rows(xr, g_ref[...], b_ref[...])
    h2_ref[...] = h2
    lg = jnp.dot(h2, wr_ref[...], preferred_element_type=jnp.float32)
    lane = lax.broadcasted_iota(jnp.int32, lg.shape, 1)
    lg = jnp.where(lane < E, lg, NEG)
    m1 = jnp.max(lg, axis=-1, keepdims=True)
    i1 = jnp.min(jnp.where(lg >= m1, lane, E), axis=-1, keepdims=True)
    lg2 = jnp.where(lane == i1, NEG, lg)
    m2 = jnp.max(lg2, axis=-1, keepdims=True)
    i2 = jnp.min(jnp.where(lg2 >= m2, lane, E), axis=-1, keepdims=True)
    w2 = 1.0 / (1.0 + jnp.exp(m1 - m2))
    i1_ref[...] = i1
    i2_ref[...] = i2
    w1_ref[...] = 1.0 - w2
    w2_ref[...] = w2


def _ln2_router(attn, wo, x, g, b, wr_pad):
    return pl.pallas_call(
        _ln2_router_body,
        grid=(NT // BR,),
        in_specs=[
            pl.BlockSpec((BR, D), lambda i: (i, 0)),
            pl.BlockSpec((D, D), lambda i: (0, 0)),
            pl.BlockSpec((BR, D), lambda i: (i, 0)),
            pl.BlockSpec((1, D), lambda i: (0, 0)),
            pl.BlockSpec((1, D), lambda i: (0, 0)),
            pl.BlockSpec((D, 128), lambda i: (0, 0)),
        ],
        out_specs=[
            pl.BlockSpec((BR, D), lambda i: (i, 0)),
            pl.BlockSpec((BR, D), lambda i: (i, 0)),
            pl.BlockSpec((BR, 1), lambda i: (i, 0)),
            pl.BlockSpec((BR, 1), lambda i: (i, 0)),
            pl.BlockSpec((BR, 1), lambda i: (i, 0)),
            pl.BlockSpec((BR, 1), lambda i: (i, 0)),
        ],
        out_shape=[
            jax.ShapeDtypeStruct((NT, D), jnp.float32),
            jax.ShapeDtypeStruct((NT, D), jnp.float32),
            jax.ShapeDtypeStruct((NT, 1), jnp.int32),
            jax.ShapeDtypeStruct((NT, 1), jnp.int32),
            jax.ShapeDtypeStruct((NT, 1), jnp.float32),
            jax.ShapeDtypeStruct((NT, 1), jnp.float32),
        ],
    )(attn, wo, x, g, b, wr_pad)


def _route_rank_body(a_ref, slot_ref, cnt_ref):
    a = a_ref[...]
    f32 = jnp.float32
    MU = (lax.broadcasted_iota(jnp.int32, (128, 128), 0) <
          lax.broadcasted_iota(jnp.int32, (128, 128), 1)).astype(f32)
    NL = (lax.broadcasted_iota(jnp.int32, (32, 32), 1) <
          lax.broadcasted_iota(jnp.int32, (32, 32), 0)).astype(f32)
    ONES = jnp.ones((128, 128), f32)
    slot = jnp.zeros(a.shape, f32)
    seg_base = jnp.zeros((), f32)
    for e in range(E):
        mask = (a == e).astype(f32)
        pre = jnp.dot(mask, MU, preferred_element_type=f32)
        rsl = jnp.dot(mask, ONES, preferred_element_type=f32)
        exr = jnp.dot(NL, rsl, preferred_element_type=f32)
        rank = pre + exr
        slot = jnp.where(a == e, seg_base + rank, slot)
        cnt = jnp.sum(mask)
        cnt_ref[e:e + 1, :] = jnp.full((1, 128), cnt, f32)
        padded = jnp.ceil(cnt * (1.0 / BM)) * BM
        seg_base = seg_base + padded
    slot_ref[...] = slot.astype(jnp.int32)


def _route_rank(aflat2d):
    return pl.pallas_call(
        _route_rank_body,
        in_specs=[pl.BlockSpec((32, 128), lambda: (0, 0))],
        out_specs=[pl.BlockSpec((32, 128), lambda: (0, 0)),
                   pl.BlockSpec((E, 128), lambda: (0, 0))],
        out_shape=[jax.ShapeDtypeStruct((32, 128), jnp.int32),
                   jax.ShapeDtypeStruct((E, 128), jnp.float32)],
        grid=(),
    )(aflat2d)


def _sc_scatter_rows(table, idx, n_out):
    from jax.experimental.pallas import tpu_sc as plsc
    NB = idx.shape[0]
    NW = 32
    b_per_w = NB // NW
    mesh = plsc.VectorSubcoreMesh(core_axis_name="c", subcore_axis_name="s")

    @functools.partial(
        pl.kernel, mesh=mesh,
        out_type=jax.ShapeDtypeStruct((n_out, D), jnp.float32),
        scratch_types=[
            pltpu.VMEM((b_per_w,), jnp.int32),
            pltpu.VMEM((b_per_w, D), jnp.float32),
            pltpu.SemaphoreType.DMA,
        ],
    )
    def k(table_hbm, idx_hbm, out_hbm, idx_v, rows_v, sem):
        wid = lax.axis_index("s") * 2 + lax.axis_index("c")
        pltpu.sync_copy(idx_hbm.at[pl.ds(wid * b_per_w, b_per_w)], idx_v)
        t0 = (wid % (NT // b_per_w)) * b_per_w
        pltpu.sync_copy(table_hbm.at[pl.ds(t0, b_per_w)], rows_v)
        pltpu.async_copy(rows_v, out_hbm.at[idx_v], sem).wait()

    return k(table, idx)


def _moe_body(te_ref, perm_ref, w1_ref, w2_ref, o_ref):
    a = jnp.dot(perm_ref[...], w1_ref[0], preferred_element_type=jnp.float32)
    a = 0.5 * a * (1.0 + lax.erf(a * (2.0 ** -0.5)))
    o_ref[...] = jnp.dot(a, w2_ref[0], preferred_element_type=jnp.float32)


def _moe_ffn(tile_expert, perm, w1, w2):
    grid_spec = pltpu.PrefetchScalarGridSpec(
        num_scalar_prefetch=1,
        grid=(P // BM,),
        in_specs=[
            pl.BlockSpec((BM, D), lambda i, te: (i, 0)),
            pl.BlockSpec((1, D, F), lambda i, te: (te[i], 0, 0)),
            pl.BlockSpec((1, F, D), lambda i, te: (te[i], 0, 0)),
        ],
        out_specs=pl.BlockSpec((BM, D), lambda i, te: (i, 0)),
    )
    return pl.pallas_call(
        _moe_body,
        grid_spec=grid_spec,
        out_shape=jax.ShapeDtypeStruct((P, D), jnp.float32),
    )(tile_expert, perm, w1, w2)


def _sc_gather(table, idx):
    from jax.experimental.pallas import tpu_sc as plsc
    NB = idx.shape[0]
    NW = 32
    b_per_w = NB // NW
    CH = 128 if b_per_w % 128 == 0 else 80
    nch = b_per_w // CH
    mesh = plsc.VectorSubcoreMesh(core_axis_name="c", subcore_axis_name="s")

    @functools.partial(
        pl.kernel, mesh=mesh,
        out_type=jax.ShapeDtypeStruct((NB, D), jnp.float32),
        scratch_types=[
            pltpu.VMEM((b_per_w,), jnp.int32),
            pltpu.VMEM((b_per_w, D), jnp.float32),
            pltpu.SemaphoreType.DMA,
        ],
    )
    def k(table_hbm, idx_hbm, out_hbm, idx_v, rows_v, sem):
        wid = lax.axis_index("s") * 2 + lax.axis_index("c")
        base = wid * b_per_w
        pltpu.sync_copy(idx_hbm.at[pl.ds(base, b_per_w)], idx_v)
        descs = [
            pltpu.async_copy(
                table_hbm.at[idx_v.at[pl.ds(j * CH, CH)]],
                rows_v.at[pl.ds(j * CH, CH)], sem)
            for j in range(nch)
        ]
        for d in descs:
            d.wait()
        pltpu.sync_copy(rows_v, out_hbm.at[pl.ds(base, b_per_w)])

    return k(table, idx)


def _combine_body(x_ref, g1_ref, g2_ref, w1_ref, w2_ref, o_ref):
    o_ref[...] = (x_ref[...] + g1_ref[...] * w1_ref[...]
                  + g2_ref[...] * w2_ref[...])


def _combine(x, g1, g2, wt1, wt2):
    return pl.pallas_call(
        _combine_body,
        grid=(NT // BR,),
        in_specs=[pl.BlockSpec((BR, D), lambda i: (i, 0))] * 3
        + [pl.BlockSpec((BR, 1), lambda i: (i, 0))] * 2,
        out_specs=pl.BlockSpec((BR, D), lambda i: (i, 0)),
        out_shape=jax.ShapeDtypeStruct((NT, D), jnp.float32),
    )(x, g1, g2, wt1, wt2)


def kernel(x, Wq, Wk, Wv, Wo, ln1_g, ln1_b, ln2_g, ln2_b, router_w, w1, w2):
    xf = x.reshape(NT, D)
    q, k, v = _ln_qkv(xf, ln1_g.reshape(1, D), ln1_b.reshape(1, D),
                      Wq, Wk, Wv)
    attn = _attention(q, k, v)

    wr_pad = jnp.zeros((D, 128), jnp.float32).at[:, :E].set(router_w.T)
    xr, h2, i1, i2, wt1, wt2 = _ln2_router(attn, Wo, xf, ln2_g.reshape(1, D),
                                           ln2_b.reshape(1, D), wr_pad)

    i32 = jnp.int32
    aflat2d = jnp.concatenate([i1, i2], axis=0).reshape(32, 128)
    slot2d, cnt2d = _route_rank(aflat2d)
    slot = slot2d.reshape(K * NT)
    counts = cnt2d[:, 0].astype(i32)
    seg_end = jnp.cumsum(((counts + BM - 1) // BM) * BM)
    ntiles = P // BM
    tile_expert = jnp.minimum(
        E - 1,
        jnp.searchsorted(seg_end, jnp.arange(ntiles, dtype=i32) * BM,
                         side='right')).astype(i32)

    perm = _sc_scatter_rows(h2, slot, P)
    eo = _moe_ffn(tile_expert, perm, w1, w2)
    g = _sc_gather(eo, slot)
    return _combine(xr, g[:NT], g[NT:], wt1, wt2).reshape(B, T, D)
